# baseline (speedup 1.0000x reference)
import functools

import jax
import jax.numpy as jnp
from jax import lax
from jax.experimental import pallas as pl
from jax.experimental.pallas import tpu as pltpu
from jax.experimental.pallas import tpu_sc as plsc

D_MODEL = 1024
SEQ = 32
ROWS = SEQ + 1
BATCH = 256
LANES = 16
BLK = ROWS * D_MODEL

_info = plsc.get_sparse_core_info()
_NC = _info.num_cores
_NS = _info.num_subcores
_NW = _NC * _NS
_BPW = BATCH // _NW

_mesh = plsc.VectorSubcoreMesh(core_axis_name="c", subcore_axis_name="s")


@functools.partial(
    pl.kernel,
    mesh=_mesh,
    out_type=jax.ShapeDtypeStruct((BATCH, ROWS, D_MODEL), jnp.float32),
    scratch_types=[
        pltpu.VMEM((BLK,), jnp.float32),
        pltpu.VMEM((_BPW, D_MODEL), jnp.float32),
        pltpu.VMEM((ROWS, D_MODEL), jnp.float32),
        pltpu.VMEM((ROWS, D_MODEL), jnp.float32),
        pltpu.SemaphoreType.DMA,
        pltpu.SemaphoreType.DMA,
        pltpu.SemaphoreType.DMA,
        pltpu.SemaphoreType.DMA,
    ],
)
def _sc_embed(x_hbm, pos_hbm, ct_hbm, out_hbm, pbuf, cbuf, ob0, ob1,
              isem0, isem1, osem0, osem1):
    wid = lax.axis_index("s") * _NC + lax.axis_index("c")
    b0 = wid * _BPW
    pltpu.sync_copy(pos_hbm, pbuf)
    pltpu.sync_copy(ct_hbm.at[pl.ds(b0, _BPW)], cbuf)

    obufs = (ob0, ob1)
    isems = (isem0, isem1)
    osems = (osem0, osem1)
    in_cp = [None] * _BPW
    out_cp = [None] * _BPW

    def start_in(i):
        in_cp[i] = pltpu.async_copy(
            x_hbm.at[b0 + i], obufs[i & 1].at[pl.ds(0, SEQ)], isems[i & 1])

    def add_block(ob, i):
        def col_body(c, cc):
            col = c * LANES
            s = pl.ds(col, LANES)
            for r in range(SEQ, 0, -1):
                ob[r, s] = ob[r - 1, s] + pbuf[pl.ds(r * D_MODEL + col, LANES)]
            ob[0, s] = cbuf[i, s] + pbuf[s]
            return cc

        lax.fori_loop(0, D_MODEL // LANES, col_body, 0)

    start_in(0)
    for i in range(_BPW):
        sl = i & 1
        if i + 1 < _BPW:
            if i - 1 >= 0:
                out_cp[i - 1].wait()
            start_in(i + 1)
        in_cp[i].wait()
        add_block(obufs[sl], i)
        out_cp[i] = pltpu.async_copy(obufs[sl], out_hbm.at[b0 + i], osems[sl])
    out_cp[_BPW - 2].wait()
    out_cp[_BPW - 1].wait()


def kernel(x, pos_table, class_token):
    return _sc_embed(
        x,
        pos_table.reshape(BLK),
        class_token.reshape(BATCH, D_MODEL),
    )


# speedup vs baseline: 1.5315x; 1.5315x over previous
import functools

import jax
import jax.numpy as jnp
from jax import lax
from jax.experimental import pallas as pl
from jax.experimental.pallas import tpu as pltpu
from jax.experimental.pallas import tpu_sc as plsc

D_MODEL = 1024
SEQ = 32
ROWS = SEQ + 1
BATCH = 256
LANES = 16
HALF = SEQ // 2
BLK = ROWS * D_MODEL

_info = plsc.get_sparse_core_info()
_NC = _info.num_cores
_NS = _info.num_subcores
_NW = _NC * _NS
_BPW = BATCH // _NW

_mesh = plsc.VectorSubcoreMesh(core_axis_name="c", subcore_axis_name="s")


@functools.partial(
    pl.kernel,
    mesh=_mesh,
    out_type=jax.ShapeDtypeStruct((BATCH, ROWS, D_MODEL), jnp.float32),
    scratch_types=[
        pltpu.VMEM((BLK,), jnp.float32),
        pltpu.VMEM((D_MODEL,), jnp.float32),
        pltpu.VMEM((D_MODEL,), jnp.float32),
        pltpu.VMEM((HALF, D_MODEL), jnp.float32),
        pltpu.VMEM((HALF, D_MODEL), jnp.float32),
        pltpu.VMEM((HALF, D_MODEL), jnp.float32),
        pltpu.VMEM((ROWS, D_MODEL), jnp.float32),
        pltpu.SemaphoreType.DMA,
        pltpu.SemaphoreType.DMA,
        pltpu.SemaphoreType.DMA,
        pltpu.SemaphoreType.DMA,
        pltpu.SemaphoreType.DMA,
        pltpu.SemaphoreType.DMA,
        pltpu.SemaphoreType.DMA,
    ],
)
def _sc_embed(x_hbm, pos_hbm, ct_hbm, out_hbm, pbuf, cb0, cb1,
              h0, h1, h2, ob, hs0, hs1, hs2, cs0, cs1, os1, os2):
    wid = lax.axis_index("s") * _NC + lax.axis_index("c")
    b0 = wid * _BPW
    pltpu.sync_copy(pos_hbm, pbuf)

    hbufs = (h0, h1, h2)
    hsems = (hs0, hs1, hs2)
    cbufs = (cb0, cb1)
    csems = (cs0, cs1)

    def start_half(i, half):
        sl = (2 * i + half) % 3
        return pltpu.async_copy(
            x_hbm.at[b0 + i, pl.ds(half * HALF, HALF)], hbufs[sl], hsems[sl])

    def start_cls(i):
        return pltpu.async_copy(ct_hbm.at[b0 + i], cbufs[i & 1], csems[i & 1])

    def add_rows(src_rows, c, col):
        s = pl.ds(col, LANES)
        xs = []
        ps = []
        for dst_r, src, src_r in src_rows:
            if src_r is None:
                xs.append(src[s])
            else:
                xs.append(src[src_r, s])
            ps.append(pbuf[pl.ds(dst_r * D_MODEL + col, LANES)])
        vs = [xv + pv for xv, pv in zip(xs, ps)]
        for (dst_r, _, _), v in zip(src_rows, vs):
            ob[dst_r, s] = v

    def chunk1(i):
        a = hbufs[(2 * i) % 3]
        cb = cbufs[i & 1]

        def body(c, cc):
            col = c * LANES
            add_rows([(0, cb, None)]
                     + [(r, a, r - 1) for r in range(1, HALF)], c, col)
            return cc

        lax.fori_loop(0, D_MODEL // LANES, body, 0)

    def chunk2(i):
        a = hbufs[(2 * i) % 3]
        b = hbufs[(2 * i + 1) % 3]

        def body(c, cc):
            col = c * LANES
            add_rows([(HALF, a, HALF - 1)]
                     + [(r, b, r - 1 - HALF) for r in range(HALF + 1, ROWS)],
                     c, col)
            return cc

        lax.fori_loop(0, D_MODEL // LANES, body, 0)

    cpA = start_half(0, 0)
    cpB = start_half(0, 1)
    cpC = start_cls(0)
    out1 = out2 = None
    for i in range(_BPW):
        cpA.wait()
        cpC.wait()
        if out1 is not None:
            out1.wait()
        chunk1(i)
        o1 = pltpu.async_copy(ob.at[pl.ds(0, HALF)],
                              out_hbm.at[b0 + i, pl.ds(0, HALF)], os1)
        if i + 1 < _BPW:
            cpC = start_cls(i + 1)
        cpB.wait()
        if out2 is not None:
            out2.wait()
        chunk2(i)
        o2 = pltpu.async_copy(ob.at[pl.ds(HALF, ROWS - HALF)],
                              out_hbm.at[b0 + i, pl.ds(HALF, ROWS - HALF)], os2)
        if i + 1 < _BPW:
            cpA = start_half(i + 1, 0)
            cpB = start_half(i + 1, 1)
        out1, out2 = o1, o2
    out1.wait()
    out2.wait()


def kernel(x, pos_table, class_token):
    return _sc_embed(
        x,
        pos_table.reshape(BLK),
        class_token.reshape(BATCH, D_MODEL),
    )


# speedup vs baseline: 2.9750x; 1.9425x over previous
import functools

import jax
import jax.numpy as jnp
from jax import lax
from jax.experimental import pallas as pl
from jax.experimental.pallas import tpu as pltpu
from jax.experimental.pallas import tpu_sc as plsc

D_MODEL = 1024
SEQ = 32
ROWS = SEQ + 1
BATCH = 256
LANES = 16
CB = 32
NTASK = ROWS * (BATCH // CB)

_info = plsc.get_sparse_core_info()
_NC = _info.num_cores
_NS = _info.num_subcores
_NW = _NC * _NS

_mesh = plsc.VectorSubcoreMesh(core_axis_name="c", subcore_axis_name="s")


@functools.partial(
    pl.kernel,
    mesh=_mesh,
    out_type=jax.ShapeDtypeStruct((ROWS, BATCH, D_MODEL), jnp.float32),
    scratch_types=[
        pltpu.VMEM((CB, D_MODEL), jnp.float32),
        pltpu.VMEM((CB, D_MODEL), jnp.float32),
        pltpu.VMEM((CB, D_MODEL), jnp.float32),
        pltpu.VMEM((D_MODEL,), jnp.float32),
        pltpu.VMEM((D_MODEL,), jnp.float32),
        pltpu.VMEM((CB,), jnp.int32),
        pltpu.VMEM((CB,), jnp.int32),
        pltpu.VMEM((CB,), jnp.int32),
        pltpu.SemaphoreType.DMA,
        pltpu.SemaphoreType.DMA,
        pltpu.SemaphoreType.DMA,
        pltpu.SemaphoreType.DMA,
        pltpu.SemaphoreType.DMA,
        pltpu.SemaphoreType.DMA,
        pltpu.SemaphoreType.DMA,
        pltpu.SemaphoreType.DMA,
    ],
)
def _sc_embed(x_hbm, pos_hbm, ct_hbm, out_hbm,
              xb0, xb1, xb2, pb0, pb1, ix0, ix1, ix2,
              sx0, sx1, sx2, sp0, sp1, so0, so1, so2):
    wid = lax.axis_index("s") * _NC + lax.axis_index("c")
    xbufs = (xb0, xb1, xb2)
    xsems = (sx0, sx1, sx2)
    osems = (so0, so1, so2)
    pbufs = (pb0, pb1)
    psems = (sp0, sp1)
    ixbufs = (ix0, ix1, ix2)

    def params(k):
        if k < 8:
            return (wid >> 3) + 4 * k, (wid & 7) * CB
        return jnp.int32(SEQ), wid * CB

    def issue_in(k):
        s, c0 = params(k)
        sl = k % 3
        pltpu.make_async_copy(pos_hbm.at[pl.ds(s * D_MODEL, D_MODEL)],
                              pbufs[k % 2], psems[k % 2]).start()

        @pl.when(s == 0)
        def _():
            pltpu.make_async_copy(ct_hbm.at[pl.ds(c0, CB)],
                                  xbufs[sl], xsems[sl]).start()

        @pl.when(s != 0)
        def _():
            ix = ixbufs[sl]
            base = c0 * SEQ + s - 1
            iota = lax.iota(jnp.int32, LANES)
            ix[pl.ds(0, LANES)] = iota * SEQ + base
            ix[pl.ds(LANES, LANES)] = iota * SEQ + base + LANES * SEQ
            pltpu.make_async_copy(x_hbm.at[ix], xbufs[sl], xsems[sl]).start()

    def wait_in(k):
        sl = k % 3
        pltpu.make_async_copy(ct_hbm.at[pl.ds(0, CB)],
                              xbufs[sl], xsems[sl]).wait()
        pltpu.make_async_copy(pos_hbm.at[pl.ds(0, D_MODEL)],
                              pbufs[k % 2], psems[k % 2]).wait()

    def compute(k):
        xb = xbufs[k % 3]
        pb = pbufs[k % 2]

        def body(c, cc):
            col = c * LANES
            s = pl.ds(col, LANES)
            vp = pb[s]
            for g in range(0, CB, 8):
                vs = [xb[r, s] for r in range(g, g + 8)]
                vs = [v + vp for v in vs]
                for r, v in zip(range(g, g + 8), vs):
                    xb[r, s] = v
            return cc

        lax.fori_loop(0, D_MODEL // LANES, body, 0)

    def out_copy(k):
        s, c0 = params(k)
        sl = k % 3
        return pltpu.make_async_copy(
            xbufs[sl], out_hbm.at[s, pl.ds(c0, CB)], osems[sl])

    issue_in(0)
    for k in range(8):
        if k >= 2:
            out_copy(k - 2).wait()
        if k + 1 < 8:
            issue_in(k + 1)
        elif k + 1 == 8:
            @pl.when(wid < 8)
            def _():
                issue_in(8)
        wait_in(k)
        compute(k)
        out_copy(k).start()
    out_copy(6).wait()
    out_copy(7).wait()

    @pl.when(wid < 8)
    def _():
        wait_in(8)
        compute(8)
        out_copy(8).start()
        out_copy(8).wait()


def kernel(x, pos_table, class_token):
    out = _sc_embed(
        x.reshape(BATCH * SEQ, D_MODEL),
        pos_table.reshape(ROWS * D_MODEL),
        class_token.reshape(BATCH, D_MODEL),
    )
    return jnp.transpose(out, (1, 0, 2))
